# baseline (speedup 1.0000x reference)
import functools

import jax
import jax.numpy as jnp
from jax import lax
from jax.experimental import pallas as pl
from jax.experimental.pallas import tpu as pltpu
from jax.experimental.pallas import tpu_sc as plsc

N = 10000
E = 320000
D = 128

NUM_CORES = 2
NUM_SUBCORES = 16
NUM_TILES = NUM_CORES * NUM_SUBCORES
K = 80
CPT = E // (NUM_TILES * K)
DCH = 400
NDCH = N // DCH
ZR = 80

BLK = 1000
GRID = N // BLK
EPS = 1e-5

_mesh = plsc.VectorSubcoreMesh(core_axis_name="c", subcore_axis_name="s")


@functools.partial(
    pl.kernel,
    out_type=[jax.ShapeDtypeStruct((N,), jnp.float32),
              jax.ShapeDtypeStruct((N,), jnp.float32)],
    mesh=_mesh,
    scratch_types=[
        pltpu.VMEM((CPT, K), jnp.int32),
        pltpu.VMEM((K,), jnp.float32),
        pltpu.VMEM((N,), jnp.float32),
        pltpu.VMEM_SHARED((N,), jnp.float32),
    ],
)
def _deg_kernel(dst_hbm, out0_hbm, out1_hbm, dstv, ones_v, zv, hist_sh):
    c = lax.axis_index("c")
    s = lax.axis_index("s")
    wid = c * NUM_SUBCORES + s

    pltpu.sync_copy(dst_hbm.at[wid], dstv)

    def fill_ones(i, carry):
        ones_v[pl.ds(i * 16, 16)] = jnp.full((16,), 1.0, jnp.float32)
        return carry
    lax.fori_loop(0, K // 16, fill_ones, 0)

    @pl.when(s == 0)
    def _():
        def zf(i, carry):
            zv[pl.ds(i * 16, 16)] = jnp.zeros((16,), jnp.float32)
            return carry
        lax.fori_loop(0, N // 16, zf, 0)
        pltpu.sync_copy(zv, hist_sh)

    plsc.subcore_barrier()

    def body(j, carry):
        pltpu.sync_copy(ones_v, hist_sh.at[dstv.at[j]], add=True)
        return carry
    lax.fori_loop(0, CPT, body, 0)

    plsc.subcore_barrier()

    @pl.when((s == 0) & (c == 0))
    def _():
        pltpu.sync_copy(hist_sh, out0_hbm)

    @pl.when((s == 0) & (c == 1))
    def _():
        pltpu.sync_copy(hist_sh, out1_hbm)


@functools.partial(
    pl.kernel,
    out_type=[jax.ShapeDtypeStruct((N, D), jnp.float32),
              jax.ShapeDtypeStruct((N, D), jnp.float32)],
    mesh=_mesh,
    scratch_types=[
        pltpu.VMEM((CPT * K,), jnp.int32),
        pltpu.VMEM((CPT, K), jnp.int32),
        pltpu.VMEM((K, D), jnp.float32),
        pltpu.VMEM((K, D), jnp.float32),
        pltpu.VMEM_SHARED((N, D), jnp.float32),
        pltpu.SemaphoreType.DMA,
        pltpu.SemaphoreType.DMA,
        pltpu.SemaphoreType.DMA,
        pltpu.SemaphoreType.DMA,
    ],
)
def _agg_kernel(g_hbm, src_hbm, dst_hbm, out0_hbm, out1_hbm, srcv, dstv,
                rows0, rows1, acc_sh, sem0, sem1, semS0, semS1):
    c = lax.axis_index("c")
    s = lax.axis_index("s")
    wid = c * NUM_SUBCORES + s

    pltpu.sync_copy(
        src_hbm.at[pl.ds(pl.multiple_of(wid * (CPT * K), 16), CPT * K)], srcv)
    pltpu.sync_copy(dst_hbm.at[wid], dstv)

    def zf(i, carry):
        rows0[i // 8, pl.ds((i % 8) * 16, 16)] = jnp.zeros((16,), jnp.float32)
        return carry
    lax.fori_loop(0, K * 8, zf, 0)

    def zero_chunk(k, carry):
        ch = s + NUM_SUBCORES * k

        @pl.when(ch < NDCH)
        def _():
            def zc(q, carry2):
                off = pl.multiple_of(ch * DCH + q * ZR, 8)
                pltpu.sync_copy(rows0, acc_sh.at[pl.ds(off, ZR)])
                return carry2
            lax.fori_loop(0, DCH // ZR, zc, 0)
        return carry
    lax.fori_loop(0, (NDCH + NUM_SUBCORES - 1) // NUM_SUBCORES, zero_chunk, 0)

    plsc.subcore_barrier()

    def _gidx(ch):
        return srcv.at[pl.ds(pl.multiple_of(ch * K, 16), K)]

    def _gwait(buf, sem):
        pltpu.make_async_copy(g_hbm.at[pl.ds(0, K)], buf, sem).wait()

    def _swait(buf, sem):
        pltpu.make_async_copy(g_hbm.at[pl.ds(0, K)], buf, sem).wait()

    pltpu.async_copy(g_hbm.at[_gidx(0)], rows0, sem0)
    pltpu.async_copy(g_hbm.at[_gidx(1)], rows1, sem1)

    def pair(i, carry):
        _gwait(rows0, sem0)
        pltpu.async_copy(rows0, acc_sh.at[dstv.at[2 * i]], semS0, add=True)
        _gwait(rows1, sem1)
        pltpu.async_copy(rows1, acc_sh.at[dstv.at[2 * i + 1]], semS1, add=True)
        _swait(rows0, semS0)
        pltpu.async_copy(g_hbm.at[_gidx(2 * i + 2)], rows0, sem0)
        _swait(rows1, semS1)
        pltpu.async_copy(g_hbm.at[_gidx(2 * i + 3)], rows1, sem1)
        return carry
    lax.fori_loop(0, (CPT - 3) // 2, pair, 0)

    _gwait(rows0, sem0)
    pltpu.async_copy(rows0, acc_sh.at[dstv.at[CPT - 3]], semS0, add=True)
    _swait(rows0, semS0)
    pltpu.async_copy(g_hbm.at[_gidx(CPT - 1)], rows0, sem0)
    _gwait(rows1, sem1)
    pltpu.async_copy(rows1, acc_sh.at[dstv.at[CPT - 2]], semS1, add=True)
    _gwait(rows0, sem0)
    pltpu.async_copy(rows0, acc_sh.at[dstv.at[CPT - 1]], semS0, add=True)
    _swait(rows1, semS1)
    _swait(rows0, semS0)

    plsc.subcore_barrier()

    def dump_chunk(k, carry):
        ch = s + NUM_SUBCORES * k

        @pl.when(ch < NDCH)
        def _():
            off = pl.multiple_of(ch * DCH, 8)

            @pl.when(c == 0)
            def _():
                pltpu.sync_copy(acc_sh.at[pl.ds(off, DCH)],
                                out0_hbm.at[pl.ds(off, DCH)])

            @pl.when(c == 1)
            def _():
                pltpu.sync_copy(acc_sh.at[pl.ds(off, DCH)],
                                out1_hbm.at[pl.ds(off, DCH)])
        return carry
    lax.fori_loop(0, (NDCH + NUM_SUBCORES - 1) // NUM_SUBCORES, dump_chunk, 0)


def _g1_body(x_ref, w_ref, d0_ref, d1_ref, g_ref, dinv_ref):
    d = d0_ref[...] + d1_ref[...]
    dinv = lax.rsqrt(1.0 + d)
    g_ref[...] = jnp.dot(x_ref[...], w_ref[...],
                         preferred_element_type=jnp.float32) * dinv
    dinv_ref[...] = dinv


_g1_call = pl.pallas_call(
    _g1_body,
    grid=(GRID,),
    in_specs=[
        pl.BlockSpec((BLK, D), lambda i: (i, 0)),
        pl.BlockSpec((D, D), lambda i: (0, 0)),
        pl.BlockSpec((BLK, 1), lambda i: (i, 0)),
        pl.BlockSpec((BLK, 1), lambda i: (i, 0)),
    ],
    out_specs=[
        pl.BlockSpec((BLK, D), lambda i: (i, 0)),
        pl.BlockSpec((BLK, 1), lambda i: (i, 0)),
    ],
    out_shape=[
        jax.ShapeDtypeStruct((N, D), jnp.float32),
        jax.ShapeDtypeStruct((N, 1), jnp.float32),
    ],
)


def _stats_body(a0_ref, a1_ref, g_ref, dinv_ref, b_ref, z_ref, s_ref):
    i = pl.program_id(0)
    a = a0_ref[...] + a1_ref[...] + g_ref[...]
    z = jnp.maximum(a * dinv_ref[...] + b_ref[...], 0.0)
    z_ref[...] = z
    cs = jnp.sum(z, axis=0, keepdims=True)
    cs2 = jnp.sum(z * z, axis=0, keepdims=True)
    st = jnp.concatenate([cs, cs2], axis=0)

    @pl.when(i == 0)
    def _():
        s_ref[...] = st

    @pl.when(i > 0)
    def _():
        s_ref[...] = s_ref[...] + st


_stats_call = pl.pallas_call(
    _stats_body,
    grid=(GRID,),
    in_specs=[
        pl.BlockSpec((BLK, D), lambda i: (i, 0)),
        pl.BlockSpec((BLK, D), lambda i: (i, 0)),
        pl.BlockSpec((BLK, D), lambda i: (i, 0)),
        pl.BlockSpec((BLK, 1), lambda i: (i, 0)),
        pl.BlockSpec((1, D), lambda i: (0, 0)),
    ],
    out_specs=[
        pl.BlockSpec((BLK, D), lambda i: (i, 0)),
        pl.BlockSpec((2, D), lambda i: (0, 0)),
    ],
    out_shape=[
        jax.ShapeDtypeStruct((N, D), jnp.float32),
        jax.ShapeDtypeStruct((2, D), jnp.float32),
    ],
)


def _gn_ln(z, st, gnw, gnb, gnms, lnw, lnb):
    mean = st[0:1] * (1.0 / N)
    ex2 = st[1:2] * (1.0 / N)
    m2 = mean * gnms
    var = ex2 - 2.0 * m2 * mean + m2 * m2
    y = gnw * (z - m2) * lax.rsqrt(var + EPS) + gnb
    rm = jnp.mean(y, axis=1, keepdims=True)
    yc = y - rm
    rv = jnp.mean(yc * yc, axis=1, keepdims=True)
    return yc * lax.rsqrt(rv + EPS) * lnw + lnb


def _mid_body(z_ref, st_ref, dinv_ref, gnw_ref, gnb_ref, gnms_ref, lnw_ref,
              lnb_ref, w2_ref, g2_ref):
    t = _gn_ln(z_ref[...], st_ref[...], gnw_ref[...], gnb_ref[...],
               gnms_ref[...], lnw_ref[...], lnb_ref[...])
    g2_ref[...] = jnp.dot(t, w2_ref[...],
                          preferred_element_type=jnp.float32) * dinv_ref[...]


_mid_call = pl.pallas_call(
    _mid_body,
    grid=(GRID,),
    in_specs=[
        pl.BlockSpec((BLK, D), lambda i: (i, 0)),
        pl.BlockSpec((2, D), lambda i: (0, 0)),
        pl.BlockSpec((BLK, 1), lambda i: (i, 0)),
    ] + [pl.BlockSpec((1, D), lambda i: (0, 0))] * 5 + [
        pl.BlockSpec((D, D), lambda i: (0, 0)),
    ],
    out_specs=pl.BlockSpec((BLK, D), lambda i: (i, 0)),
    out_shape=jax.ShapeDtypeStruct((N, D), jnp.float32),
)


def _final_body(z_ref, st_ref, gnw_ref, gnb_ref, gnms_ref, lnw_ref, lnb_ref,
                fcw_ref, fcb_ref, emb_ref, pooled):
    i = pl.program_id(0)
    t = _gn_ln(z_ref[...], st_ref[...], gnw_ref[...], gnb_ref[...],
               gnms_ref[...], lnw_ref[...], lnb_ref[...])
    bm = jnp.max(t, axis=0, keepdims=True)

    @pl.when(i == 0)
    def _():
        pooled[...] = bm

    @pl.when(i > 0)
    def _():
        pooled[...] = jnp.maximum(pooled[...], bm)

    @pl.when(i == GRID - 1)
    def _():
        emb_ref[...] = lax.dot_general(
            pooled[...], fcw_ref[...], (((1,), (1,)), ((), ())),
            preferred_element_type=jnp.float32) + fcb_ref[...]


_final_call = pl.pallas_call(
    _final_body,
    grid=(GRID,),
    in_specs=[
        pl.BlockSpec((BLK, D), lambda i: (i, 0)),
        pl.BlockSpec((2, D), lambda i: (0, 0)),
    ] + [pl.BlockSpec((1, D), lambda i: (0, 0))] * 5 + [
        pl.BlockSpec((D, D), lambda i: (0, 0)),
        pl.BlockSpec((1, D), lambda i: (0, 0)),
    ],
    out_specs=pl.BlockSpec((1, D), lambda i: (0, 0)),
    out_shape=jax.ShapeDtypeStruct((1, D), jnp.float32),
    scratch_shapes=[pltpu.VMEM((1, D), jnp.float32)],
)


def kernel(x, edge_index, W1, b1, W2, b2, gn_weight, gn_bias, gn_mean_scale,
           ln_weight, ln_bias, fc_W, fc_b):
    src_flat = edge_index[0]
    dst_r = edge_index[1].reshape(NUM_TILES, CPT, K)

    gnw = gn_weight.reshape(1, D)
    gnb = gn_bias.reshape(1, D)
    gnms = gn_mean_scale.reshape(1, D)
    lnw = ln_weight.reshape(1, D)
    lnb = ln_bias.reshape(1, D)

    d0, d1 = _deg_kernel(dst_r)
    d0 = d0.reshape(N, 1)
    d1 = d1.reshape(N, 1)

    g1, dinv = _g1_call(x, W1, d0, d1)
    a0, a1 = _agg_kernel(g1, src_flat, dst_r)
    z1, st1 = _stats_call(a0, a1, g1, dinv, b1.reshape(1, D))
    g2 = _mid_call(z1, st1, dinv, gnw, gnb, gnms, lnw, lnb, W2)
    a0, a1 = _agg_kernel(g2, src_flat, dst_r)
    z2, st2 = _stats_call(a0, a1, g2, dinv, b2.reshape(1, D))
    emb = _final_call(z2, st2, gnw, gnb, gnms, lnw, lnb, fc_W,
                      fc_b.reshape(1, D))
    return emb.reshape(D)


# speedup vs baseline: 1.2301x; 1.2301x over previous
import functools

import jax
import jax.numpy as jnp
from jax import lax
from jax.experimental import pallas as pl
from jax.experimental.pallas import tpu as pltpu
from jax.experimental.pallas import tpu_sc as plsc

N = 10000
E = 320000
D = 128

NUM_CORES = 2
NUM_SUBCORES = 16
NUM_TILES = NUM_CORES * NUM_SUBCORES
K = 80
CPT = E // (NUM_TILES * K)
DCH = 400
NDCH = N // DCH
ZR = 80

BLK = 2000
GRID = N // BLK
EPS = 1e-5

_mesh = plsc.VectorSubcoreMesh(core_axis_name="c", subcore_axis_name="s")


@functools.partial(
    pl.kernel,
    out_type=[jax.ShapeDtypeStruct((N,), jnp.float32),
              jax.ShapeDtypeStruct((N,), jnp.float32)],
    mesh=_mesh,
    scratch_types=[
        pltpu.VMEM((CPT, K), jnp.int32),
        pltpu.VMEM((K,), jnp.float32),
        pltpu.VMEM((N,), jnp.float32),
        pltpu.VMEM_SHARED((N,), jnp.float32),
    ],
)
def _deg_kernel(dst_hbm, out0_hbm, out1_hbm, dstv, ones_v, zv, hist_sh):
    c = lax.axis_index("c")
    s = lax.axis_index("s")
    wid = c * NUM_SUBCORES + s

    pltpu.sync_copy(dst_hbm.at[wid], dstv)

    def fill_ones(i, carry):
        ones_v[pl.ds(i * 16, 16)] = jnp.full((16,), 1.0, jnp.float32)
        return carry
    lax.fori_loop(0, K // 16, fill_ones, 0)

    @pl.when(s == 0)
    def _():
        def zf(i, carry):
            zv[pl.ds(i * 16, 16)] = jnp.zeros((16,), jnp.float32)
            return carry
        lax.fori_loop(0, N // 16, zf, 0)
        pltpu.sync_copy(zv, hist_sh)

    plsc.subcore_barrier()

    def body(j, carry):
        pltpu.sync_copy(ones_v, hist_sh.at[dstv.at[j]], add=True)
        return carry
    lax.fori_loop(0, CPT, body, 0)

    plsc.subcore_barrier()

    @pl.when((s == 0) & (c == 0))
    def _():
        pltpu.sync_copy(hist_sh, out0_hbm)

    @pl.when((s == 0) & (c == 1))
    def _():
        pltpu.sync_copy(hist_sh, out1_hbm)


@functools.partial(
    pl.kernel,
    out_type=[jax.ShapeDtypeStruct((N, D), jnp.float32),
              jax.ShapeDtypeStruct((N, D), jnp.float32)],
    mesh=_mesh,
    scratch_types=[
        pltpu.VMEM((CPT * K,), jnp.int32),
        pltpu.VMEM((CPT, K), jnp.int32),
        pltpu.VMEM((K, D), jnp.float32),
        pltpu.VMEM((K, D), jnp.float32),
        pltpu.VMEM_SHARED((N, D), jnp.float32),
        pltpu.SemaphoreType.DMA,
        pltpu.SemaphoreType.DMA,
    ],
)
def _agg_kernel(g_hbm, src_hbm, dst_hbm, out0_hbm, out1_hbm, srcv, dstv,
                rows0, rows1, acc_sh, sem0, sem1):
    c = lax.axis_index("c")
    s = lax.axis_index("s")
    wid = c * NUM_SUBCORES + s

    pltpu.sync_copy(
        src_hbm.at[pl.ds(pl.multiple_of(wid * (CPT * K), 16), CPT * K)], srcv)
    pltpu.sync_copy(dst_hbm.at[wid], dstv)

    def zf(i, carry):
        rows0[i // 8, pl.ds((i % 8) * 16, 16)] = jnp.zeros((16,), jnp.float32)
        return carry
    lax.fori_loop(0, K * 8, zf, 0)

    def zero_chunk(k, carry):
        ch = s + NUM_SUBCORES * k

        @pl.when(ch < NDCH)
        def _():
            def zc(q, carry2):
                off = pl.multiple_of(ch * DCH + q * ZR, 8)
                pltpu.sync_copy(rows0, acc_sh.at[pl.ds(off, ZR)])
                return carry2
            lax.fori_loop(0, DCH // ZR, zc, 0)
        return carry
    lax.fori_loop(0, (NDCH + NUM_SUBCORES - 1) // NUM_SUBCORES, zero_chunk, 0)

    plsc.subcore_barrier()

    def _gidx(ch):
        return srcv.at[pl.ds(pl.multiple_of(ch * K, 16), K)]

    def _gwait(buf, sem):
        pltpu.make_async_copy(g_hbm.at[pl.ds(0, K)], buf, sem).wait()

    pltpu.async_copy(g_hbm.at[_gidx(0)], rows0, sem0)

    def pair(i, carry):
        pltpu.async_copy(g_hbm.at[_gidx(2 * i + 1)], rows1, sem1)
        _gwait(rows0, sem0)
        pltpu.sync_copy(rows0, acc_sh.at[dstv.at[2 * i]], add=True)
        pltpu.async_copy(g_hbm.at[_gidx(2 * i + 2)], rows0, sem0)
        _gwait(rows1, sem1)
        pltpu.sync_copy(rows1, acc_sh.at[dstv.at[2 * i + 1]], add=True)
        return carry
    lax.fori_loop(0, (CPT - 1) // 2, pair, 0)

    _gwait(rows0, sem0)
    pltpu.sync_copy(rows0, acc_sh.at[dstv.at[CPT - 1]], add=True)

    plsc.subcore_barrier()

    def dump_chunk(k, carry):
        ch = s + NUM_SUBCORES * k

        @pl.when(ch < NDCH)
        def _():
            off = pl.multiple_of(ch * DCH, 8)

            @pl.when(c == 0)
            def _():
                pltpu.sync_copy(acc_sh.at[pl.ds(off, DCH)],
                                out0_hbm.at[pl.ds(off, DCH)])

            @pl.when(c == 1)
            def _():
                pltpu.sync_copy(acc_sh.at[pl.ds(off, DCH)],
                                out1_hbm.at[pl.ds(off, DCH)])
        return carry
    lax.fori_loop(0, (NDCH + NUM_SUBCORES - 1) // NUM_SUBCORES, dump_chunk, 0)


def _g1_body(x_ref, w_ref, d0_ref, d1_ref, g_ref, dinv_ref):
    d = d0_ref[...] + d1_ref[...]
    dinv = lax.rsqrt(1.0 + d)
    g_ref[...] = jnp.dot(x_ref[...], w_ref[...],
                         preferred_element_type=jnp.float32) * dinv
    dinv_ref[...] = dinv


_g1_call = pl.pallas_call(
    _g1_body,
    grid=(GRID,),
    in_specs=[
        pl.BlockSpec((BLK, D), lambda i: (i, 0)),
        pl.BlockSpec((D, D), lambda i: (0, 0)),
        pl.BlockSpec((BLK, 1), lambda i: (i, 0)),
        pl.BlockSpec((BLK, 1), lambda i: (i, 0)),
    ],
    out_specs=[
        pl.BlockSpec((BLK, D), lambda i: (i, 0)),
        pl.BlockSpec((BLK, 1), lambda i: (i, 0)),
    ],
    out_shape=[
        jax.ShapeDtypeStruct((N, D), jnp.float32),
        jax.ShapeDtypeStruct((N, 1), jnp.float32),
    ],
)


def _stats_body(a0_ref, a1_ref, g_ref, dinv_ref, b_ref, z_ref, s_ref):
    i = pl.program_id(0)
    a = a0_ref[...] + a1_ref[...] + g_ref[...]
    z = jnp.maximum(a * dinv_ref[...] + b_ref[...], 0.0)
    z_ref[...] = z
    cs = jnp.sum(z, axis=0, keepdims=True)
    cs2 = jnp.sum(z * z, axis=0, keepdims=True)
    st = jnp.concatenate([cs, cs2], axis=0)

    @pl.when(i == 0)
    def _():
        s_ref[...] = st

    @pl.when(i > 0)
    def _():
        s_ref[...] = s_ref[...] + st


_stats_call = pl.pallas_call(
    _stats_body,
    grid=(GRID,),
    in_specs=[
        pl.BlockSpec((BLK, D), lambda i: (i, 0)),
        pl.BlockSpec((BLK, D), lambda i: (i, 0)),
        pl.BlockSpec((BLK, D), lambda i: (i, 0)),
        pl.BlockSpec((BLK, 1), lambda i: (i, 0)),
        pl.BlockSpec((1, D), lambda i: (0, 0)),
    ],
    out_specs=[
        pl.BlockSpec((BLK, D), lambda i: (i, 0)),
        pl.BlockSpec((2, D), lambda i: (0, 0)),
    ],
    out_shape=[
        jax.ShapeDtypeStruct((N, D), jnp.float32),
        jax.ShapeDtypeStruct((2, D), jnp.float32),
    ],
)


def _gn_ln(z, st, gnw, gnb, gnms, lnw, lnb):
    mean = st[0:1] * (1.0 / N)
    ex2 = st[1:2] * (1.0 / N)
    m2 = mean * gnms
    var = ex2 - 2.0 * m2 * mean + m2 * m2
    y = gnw * (z - m2) * lax.rsqrt(var + EPS) + gnb
    rm = jnp.mean(y, axis=1, keepdims=True)
    yc = y - rm
    rv = jnp.mean(yc * yc, axis=1, keepdims=True)
    return yc * lax.rsqrt(rv + EPS) * lnw + lnb


def _mid_body(z_ref, st_ref, dinv_ref, gnw_ref, gnb_ref, gnms_ref, lnw_ref,
              lnb_ref, w2_ref, g2_ref):
    t = _gn_ln(z_ref[...], st_ref[...], gnw_ref[...], gnb_ref[...],
               gnms_ref[...], lnw_ref[...], lnb_ref[...])
    g2_ref[...] = jnp.dot(t, w2_ref[...],
                          preferred_element_type=jnp.float32) * dinv_ref[...]


_mid_call = pl.pallas_call(
    _mid_body,
    grid=(GRID,),
    in_specs=[
        pl.BlockSpec((BLK, D), lambda i: (i, 0)),
        pl.BlockSpec((2, D), lambda i: (0, 0)),
        pl.BlockSpec((BLK, 1), lambda i: (i, 0)),
    ] + [pl.BlockSpec((1, D), lambda i: (0, 0))] * 5 + [
        pl.BlockSpec((D, D), lambda i: (0, 0)),
    ],
    out_specs=pl.BlockSpec((BLK, D), lambda i: (i, 0)),
    out_shape=jax.ShapeDtypeStruct((N, D), jnp.float32),
)


def _final_body(z_ref, st_ref, gnw_ref, gnb_ref, gnms_ref, lnw_ref, lnb_ref,
                fcw_ref, fcb_ref, emb_ref, pooled):
    i = pl.program_id(0)
    t = _gn_ln(z_ref[...], st_ref[...], gnw_ref[...], gnb_ref[...],
               gnms_ref[...], lnw_ref[...], lnb_ref[...])
    bm = jnp.max(t, axis=0, keepdims=True)

    @pl.when(i == 0)
    def _():
        pooled[...] = bm

    @pl.when(i > 0)
    def _():
        pooled[...] = jnp.maximum(pooled[...], bm)

    @pl.when(i == GRID - 1)
    def _():
        emb_ref[...] = lax.dot_general(
            pooled[...], fcw_ref[...], (((1,), (1,)), ((), ())),
            preferred_element_type=jnp.float32) + fcb_ref[...]


_final_call = pl.pallas_call(
    _final_body,
    grid=(GRID,),
    in_specs=[
        pl.BlockSpec((BLK, D), lambda i: (i, 0)),
        pl.BlockSpec((2, D), lambda i: (0, 0)),
    ] + [pl.BlockSpec((1, D), lambda i: (0, 0))] * 5 + [
        pl.BlockSpec((D, D), lambda i: (0, 0)),
        pl.BlockSpec((1, D), lambda i: (0, 0)),
    ],
    out_specs=pl.BlockSpec((1, D), lambda i: (0, 0)),
    out_shape=jax.ShapeDtypeStruct((1, D), jnp.float32),
    scratch_shapes=[pltpu.VMEM((1, D), jnp.float32)],
)


def kernel(x, edge_index, W1, b1, W2, b2, gn_weight, gn_bias, gn_mean_scale,
           ln_weight, ln_bias, fc_W, fc_b):
    src_flat = edge_index[0]
    dst_r = edge_index[1].reshape(NUM_TILES, CPT, K)

    gnw = gn_weight.reshape(1, D)
    gnb = gn_bias.reshape(1, D)
    gnms = gn_mean_scale.reshape(1, D)
    lnw = ln_weight.reshape(1, D)
    lnb = ln_bias.reshape(1, D)

    d0, d1 = _deg_kernel(dst_r)
    d0 = d0.reshape(N, 1)
    d1 = d1.reshape(N, 1)

    g1, dinv = _g1_call(x, W1, d0, d1)
    a0, a1 = _agg_kernel(g1, src_flat, dst_r)
    z1, st1 = _stats_call(a0, a1, g1, dinv, b1.reshape(1, D))
    g2 = _mid_call(z1, st1, dinv, gnw, gnb, gnms, lnw, lnb, W2)
    a0, a1 = _agg_kernel(g2, src_flat, dst_r)
    z2, st2 = _stats_call(a0, a1, g2, dinv, b2.reshape(1, D))
    emb = _final_call(z2, st2, gnw, gnb, gnms, lnw, lnb, fc_W,
                      fc_b.reshape(1, D))
    return emb.reshape(D)


# speedup vs baseline: 1.3812x; 1.1228x over previous
import functools

import jax
import jax.numpy as jnp
from jax import lax
from jax.experimental import pallas as pl
from jax.experimental.pallas import tpu as pltpu
from jax.experimental.pallas import tpu_sc as plsc

N = 10000
E = 320000
D = 128

NUM_CORES = 2
NUM_SUBCORES = 16
NUM_TILES = NUM_CORES * NUM_SUBCORES
K = 80
CPT = 128
EPT = CPT * K
NPAD = EPT - E // NUM_TILES
NJ = 16
GSZ = 32
NG = CPT // GSZ
DCH = 400
NDCH = N // DCH
ZR = 80

BLK = 2000
GRID = N // BLK
EPS = 1e-5

_mesh = plsc.VectorSubcoreMesh(core_axis_name="c", subcore_axis_name="s")


@functools.partial(
    pl.kernel,
    out_type=[jax.ShapeDtypeStruct((N + NJ,), jnp.float32),
              jax.ShapeDtypeStruct((N + NJ,), jnp.float32)],
    mesh=_mesh,
    scratch_types=[
        pltpu.VMEM((CPT, K), jnp.int32),
        pltpu.VMEM((K,), jnp.float32),
        pltpu.VMEM((N + NJ,), jnp.float32),
        pltpu.VMEM_SHARED((N + NJ,), jnp.float32),
    ],
)
def _deg_kernel(dst_hbm, out0_hbm, out1_hbm, dstv, ones_v, zv, hist_sh):
    c = lax.axis_index("c")
    s = lax.axis_index("s")
    wid = c * NUM_SUBCORES + s

    pltpu.sync_copy(dst_hbm.at[wid], dstv)

    def fill_ones(i, carry):
        ones_v[pl.ds(i * 16, 16)] = jnp.full((16,), 1.0, jnp.float32)
        return carry
    lax.fori_loop(0, K // 16, fill_ones, 0)

    @pl.when(s == 0)
    def _():
        def zf(i, carry):
            zv[pl.ds(i * 16, 16)] = jnp.zeros((16,), jnp.float32)
            return carry
        lax.fori_loop(0, (N + NJ) // 16, zf, 0)
        pltpu.sync_copy(zv, hist_sh)

    plsc.subcore_barrier()

    def body(j, carry):
        pltpu.sync_copy(ones_v, hist_sh.at[dstv.at[j]], add=True)
        return carry
    lax.fori_loop(0, CPT, body, 0)

    plsc.subcore_barrier()

    @pl.when((s == 0) & (c == 0))
    def _():
        pltpu.sync_copy(hist_sh, out0_hbm)

    @pl.when((s == 0) & (c == 1))
    def _():
        pltpu.sync_copy(hist_sh, out1_hbm)


@functools.partial(
    pl.kernel,
    out_type=[jax.ShapeDtypeStruct((N, D), jnp.float32),
              jax.ShapeDtypeStruct((N, D), jnp.float32)],
    mesh=_mesh,
    scratch_types=[
        pltpu.VMEM((EPT,), jnp.int32),
        pltpu.VMEM((GSZ, K), jnp.int32),
        pltpu.VMEM((GSZ, K), jnp.int32),
        pltpu.VMEM((K, D), jnp.float32),
        pltpu.VMEM((K, D), jnp.float32),
        pltpu.VMEM((K, D), jnp.float32),
        pltpu.VMEM_SHARED((N + NJ, D), jnp.float32),
        pltpu.SemaphoreType.DMA,
        pltpu.SemaphoreType.DMA,
        pltpu.SemaphoreType.DMA,
    ],
)
def _agg_kernel(g_hbm, src_hbm, dst_hbm, out0_hbm, out1_hbm, srcv, dstR0,
                dstR1, rows0, rows1, rows2, acc_sh, sem0, sem1, sem2):
    c = lax.axis_index("c")
    s = lax.axis_index("s")
    wid = c * NUM_SUBCORES + s
    bufs = (rows0, rows1, rows2)
    sems = (sem0, sem1, sem2)
    rings = (dstR0, dstR1)

    pltpu.sync_copy(src_hbm.at[pl.ds(pl.multiple_of(wid * EPT, 16), EPT)],
                    srcv)
    pltpu.sync_copy(dst_hbm.at[wid, pl.ds(0, GSZ)], dstR0)

    def zf(i, carry):
        rows0[i // 8, pl.ds((i % 8) * 16, 16)] = jnp.zeros((16,), jnp.float32)
        return carry
    lax.fori_loop(0, K * 8, zf, 0)

    def zero_chunk(k, carry):
        ch = s + NUM_SUBCORES * k

        @pl.when(ch < NDCH)
        def _():
            def zc(q, carry2):
                off = pl.multiple_of(ch * DCH + q * ZR, 8)
                pltpu.sync_copy(rows0, acc_sh.at[pl.ds(off, ZR)])
                return carry2
            lax.fori_loop(0, DCH // ZR, zc, 0)
        return carry
    lax.fori_loop(0, (NDCH + NUM_SUBCORES - 1) // NUM_SUBCORES, zero_chunk, 0)

    plsc.subcore_barrier()

    def _gidx(ch):
        return srcv.at[pl.ds(pl.multiple_of(ch * K, 16), K)]

    def _gwait(buf, sem):
        pltpu.make_async_copy(g_hbm.at[pl.ds(0, K)], buf, sem).wait()

    pltpu.async_copy(g_hbm.at[_gidx(0)], rows0, sem0)
    pltpu.async_copy(g_hbm.at[_gidx(1)], rows1, sem1)
    pltpu.async_copy(g_hbm.at[_gidx(2)], rows2, sem2)

    def group(gi, carry):
        for pp in range(2):
            @pl.when((gi < NG - 1) & (lax.rem(gi + 1, 2) == pp))
            def _():
                pltpu.sync_copy(
                    dst_hbm.at[wid, pl.ds(pl.multiple_of((gi + 1) * GSZ, 8),
                                          GSZ)], rings[pp])

        def inner(q, carry2):
            ch = gi * GSZ + q
            b = lax.rem(ch, 3)
            for bb in range(3):
                @pl.when(b == bb)
                def _():
                    _gwait(bufs[bb], sems[bb])
                    for pp in range(2):
                        @pl.when(lax.rem(gi, 2) == pp)
                        def _():
                            pltpu.sync_copy(bufs[bb],
                                            acc_sh.at[rings[pp].at[q]],
                                            add=True)

                    @pl.when(ch + 3 < CPT)
                    def _():
                        pltpu.async_copy(g_hbm.at[_gidx(ch + 3)], bufs[bb],
                                         sems[bb])
            return carry2
        lax.fori_loop(0, GSZ, inner, 0)
        return carry
    lax.fori_loop(0, NG, group, 0)

    plsc.subcore_barrier()

    def dump_chunk(k, carry):
        ch = s + NUM_SUBCORES * k

        @pl.when(ch < NDCH)
        def _():
            off = pl.multiple_of(ch * DCH, 8)

            @pl.when(c == 0)
            def _():
                pltpu.sync_copy(acc_sh.at[pl.ds(off, DCH)],
                                out0_hbm.at[pl.ds(off, DCH)])

            @pl.when(c == 1)
            def _():
                pltpu.sync_copy(acc_sh.at[pl.ds(off, DCH)],
                                out1_hbm.at[pl.ds(off, DCH)])
        return carry
    lax.fori_loop(0, (NDCH + NUM_SUBCORES - 1) // NUM_SUBCORES, dump_chunk, 0)


def _g1_body(x_ref, w_ref, d0_ref, d1_ref, g_ref, dinv_ref):
    d = d0_ref[...] + d1_ref[...]
    dinv = lax.rsqrt(1.0 + d)
    g_ref[...] = jnp.dot(x_ref[...], w_ref[...],
                         preferred_element_type=jnp.float32) * dinv
    dinv_ref[...] = dinv


_g1_call = pl.pallas_call(
    _g1_body,
    grid=(GRID,),
    in_specs=[
        pl.BlockSpec((BLK, D), lambda i: (i, 0)),
        pl.BlockSpec((D, D), lambda i: (0, 0)),
        pl.BlockSpec((BLK, 1), lambda i: (i, 0)),
        pl.BlockSpec((BLK, 1), lambda i: (i, 0)),
    ],
    out_specs=[
        pl.BlockSpec((BLK, D), lambda i: (i, 0)),
        pl.BlockSpec((BLK, 1), lambda i: (i, 0)),
    ],
    out_shape=[
        jax.ShapeDtypeStruct((N, D), jnp.float32),
        jax.ShapeDtypeStruct((N, 1), jnp.float32),
    ],
)


def _stats_body(a0_ref, a1_ref, g_ref, dinv_ref, b_ref, z_ref, s_ref):
    i = pl.program_id(0)
    a = a0_ref[...] + a1_ref[...] + g_ref[...]
    z = jnp.maximum(a * dinv_ref[...] + b_ref[...], 0.0)
    z_ref[...] = z
    cs = jnp.sum(z, axis=0, keepdims=True)
    cs2 = jnp.sum(z * z, axis=0, keepdims=True)
    st = jnp.concatenate([cs, cs2], axis=0)

    @pl.when(i == 0)
    def _():
        s_ref[...] = st

    @pl.when(i > 0)
    def _():
        s_ref[...] = s_ref[...] + st


_stats_call = pl.pallas_call(
    _stats_body,
    grid=(GRID,),
    in_specs=[
        pl.BlockSpec((BLK, D), lambda i: (i, 0)),
        pl.BlockSpec((BLK, D), lambda i: (i, 0)),
        pl.BlockSpec((BLK, D), lambda i: (i, 0)),
        pl.BlockSpec((BLK, 1), lambda i: (i, 0)),
        pl.BlockSpec((1, D), lambda i: (0, 0)),
    ],
    out_specs=[
        pl.BlockSpec((BLK, D), lambda i: (i, 0)),
        pl.BlockSpec((2, D), lambda i: (0, 0)),
    ],
    out_shape=[
        jax.ShapeDtypeStruct((N, D), jnp.float32),
        jax.ShapeDtypeStruct((2, D), jnp.float32),
    ],
)


def _gn_ln(z, st, gnw, gnb, gnms, lnw, lnb):
    mean = st[0:1] * (1.0 / N)
    ex2 = st[1:2] * (1.0 / N)
    m2 = mean * gnms
    var = ex2 - 2.0 * m2 * mean + m2 * m2
    y = gnw * (z - m2) * lax.rsqrt(var + EPS) + gnb
    rm = jnp.mean(y, axis=1, keepdims=True)
    yc = y - rm
    rv = jnp.mean(yc * yc, axis=1, keepdims=True)
    return yc * lax.rsqrt(rv + EPS) * lnw + lnb


def _mid_body(z_ref, st_ref, dinv_ref, gnw_ref, gnb_ref, gnms_ref, lnw_ref,
              lnb_ref, w2_ref, g2_ref):
    t = _gn_ln(z_ref[...], st_ref[...], gnw_ref[...], gnb_ref[...],
               gnms_ref[...], lnw_ref[...], lnb_ref[...])
    g2_ref[...] = jnp.dot(t, w2_ref[...],
                          preferred_element_type=jnp.float32) * dinv_ref[...]


_mid_call = pl.pallas_call(
    _mid_body,
    grid=(GRID,),
    in_specs=[
        pl.BlockSpec((BLK, D), lambda i: (i, 0)),
        pl.BlockSpec((2, D), lambda i: (0, 0)),
        pl.BlockSpec((BLK, 1), lambda i: (i, 0)),
    ] + [pl.BlockSpec((1, D), lambda i: (0, 0))] * 5 + [
        pl.BlockSpec((D, D), lambda i: (0, 0)),
    ],
    out_specs=pl.BlockSpec((BLK, D), lambda i: (i, 0)),
    out_shape=jax.ShapeDtypeStruct((N, D), jnp.float32),
)


def _final_body(z_ref, st_ref, gnw_ref, gnb_ref, gnms_ref, lnw_ref, lnb_ref,
                fcw_ref, fcb_ref, emb_ref, pooled):
    i = pl.program_id(0)
    t = _gn_ln(z_ref[...], st_ref[...], gnw_ref[...], gnb_ref[...],
               gnms_ref[...], lnw_ref[...], lnb_ref[...])
    bm = jnp.max(t, axis=0, keepdims=True)

    @pl.when(i == 0)
    def _():
        pooled[...] = bm

    @pl.when(i > 0)
    def _():
        pooled[...] = jnp.maximum(pooled[...], bm)

    @pl.when(i == GRID - 1)
    def _():
        emb_ref[...] = lax.dot_general(
            pooled[...], fcw_ref[...], (((1,), (1,)), ((), ())),
            preferred_element_type=jnp.float32) + fcb_ref[...]


_final_call = pl.pallas_call(
    _final_body,
    grid=(GRID,),
    in_specs=[
        pl.BlockSpec((BLK, D), lambda i: (i, 0)),
        pl.BlockSpec((2, D), lambda i: (0, 0)),
    ] + [pl.BlockSpec((1, D), lambda i: (0, 0))] * 5 + [
        pl.BlockSpec((D, D), lambda i: (0, 0)),
        pl.BlockSpec((1, D), lambda i: (0, 0)),
    ],
    out_specs=pl.BlockSpec((1, D), lambda i: (0, 0)),
    out_shape=jax.ShapeDtypeStruct((1, D), jnp.float32),
    scratch_shapes=[pltpu.VMEM((1, D), jnp.float32)],
)


def kernel(x, edge_index, W1, b1, W2, b2, gn_weight, gn_bias, gn_mean_scale,
           ln_weight, ln_bias, fc_W, fc_b):
    pad_src = (jnp.arange(NUM_TILES * NPAD, dtype=jnp.int32) * 37) % N
    pad_dst = N + (jnp.arange(NUM_TILES * NPAD, dtype=jnp.int32) % NJ)
    src_flat = jnp.concatenate(
        [edge_index[0].reshape(NUM_TILES, -1),
         pad_src.reshape(NUM_TILES, NPAD)], axis=1).reshape(-1)
    dst_r = jnp.concatenate(
        [edge_index[1].reshape(NUM_TILES, -1),
         pad_dst.reshape(NUM_TILES, NPAD)], axis=1).reshape(NUM_TILES, CPT, K)

    gnw = gn_weight.reshape(1, D)
    gnb = gn_bias.reshape(1, D)
    gnms = gn_mean_scale.reshape(1, D)
    lnw = ln_weight.reshape(1, D)
    lnb = ln_bias.reshape(1, D)

    d0, d1 = _deg_kernel(dst_r)
    d0 = d0[:N].reshape(N, 1)
    d1 = d1[:N].reshape(N, 1)

    g1, dinv = _g1_call(x, W1, d0, d1)
    a0, a1 = _agg_kernel(g1, src_flat, dst_r)
    z1, st1 = _stats_call(a0, a1, g1, dinv, b1.reshape(1, D))
    g2 = _mid_call(z1, st1, dinv, gnw, gnb, gnms, lnw, lnb, W2)
    a0, a1 = _agg_kernel(g2, src_flat, dst_r)
    z2, st2 = _stats_call(a0, a1, g2, dinv, b2.reshape(1, D))
    emb = _final_call(z2, st2, gnw, gnb, gnms, lnw, lnb, fc_W,
                      fc_b.reshape(1, D))
    return emb.reshape(D)
